# baseline (speedup 1.0000x reference)
import functools

import jax
import jax.numpy as jnp
from jax import lax
from jax.experimental import pallas as pl
from jax.experimental.pallas import tpu as pltpu
from jax.experimental.pallas import tpu_sc as plsc

N_SUB = 16
GROUP = 512
FDIM = 128
NEDGE = 131072
NNODES = N_SUB * GROUP
A_SIZE = NNODES * GROUP
A_PAD = 8
DUMP = A_SIZE
WIN = 128
HALF_WORDS = A_SIZE // 2
TILE_WORDS = HALF_WORDS // 16
ZCHUNK = 16384


def _sc_build_adj(src, dst):
    mesh = plsc.VectorSubcoreMesh(core_axis_name="core", subcore_axis_name="subcore")

    @functools.partial(
        pl.kernel,
        out_type=jax.ShapeDtypeStruct((A_SIZE + A_PAD,), jnp.float32),
        mesh=mesh,
        scratch_types=[
            pltpu.VMEM((ZCHUNK,), jnp.float32),
            pltpu.VMEM((WIN,), jnp.float32),
            pltpu.VMEM((1, WIN), jnp.int32),
        ],
    )
    def build(src_hbm, dst_hbm, a_hbm, zbuf, ones, idxbuf):
        cid = lax.axis_index("core")
        sid = lax.axis_index("subcore")

        @pl.loop(0, ZCHUNK, step=16)
        def _(i):
            zbuf[pl.ds(i, 16)] = jnp.zeros((16,), jnp.float32)

        @pl.loop(0, WIN, step=16)
        def _(i):
            ones[pl.ds(i, 16)] = jnp.full((16,), 1.0, jnp.float32)

        base = cid * HALF_WORDS + sid * TILE_WORDS

        @pl.loop(0, TILE_WORDS // ZCHUNK)
        def _(j):
            pltpu.sync_copy(zbuf, a_hbm.at[pl.ds(base + j * ZCHUNK, ZCHUNK)])

        plsc.subcore_barrier()

        def body(s_vmem, d_vmem):
            for c in range(WIN // 16):
                sl = pl.ds(c * 16, 16)
                sv = s_vmem[0, sl]
                dv = d_vmem[0, sl]
                same_block = (sv >> 9) == (dv >> 9)
                mine = (sv >> 12) == cid
                flat = sv * GROUP + (dv & (GROUP - 1))
                idxbuf[0, sl] = jnp.where(same_block & mine, flat, DUMP)
            pltpu.sync_copy(ones, a_hbm.at[idxbuf.at[0]])

        pltpu.emit_pipeline(
            body,
            grid=(NEDGE // WIN,),
            in_specs=[
                pl.BlockSpec((1, WIN), index_map=lambda i: (0, i)),
                pl.BlockSpec((1, WIN), index_map=lambda i: (0, i)),
            ],
            out_specs=[],
            core_axis_name="subcore",
            dimension_semantics=(pltpu.PARALLEL,),
        )(src_hbm, dst_hbm)

    return build(src, dst)


def _tc_pool_body(x_ref, a_ref, w_ref, b_ref, out_ref, diag_ref):
    x = x_ref[0]
    w = w_ref[...]
    b = b_ref[...]
    logits = jnp.dot(x, w, preferred_element_type=jnp.float32) + b
    m = jnp.max(logits, axis=1, keepdims=True)
    e = jnp.exp(logits - m)
    s = e / jnp.sum(e, axis=1, keepdims=True)
    out_ref[0] = lax.dot_general(
        s, x, (((0,), (0,)), ((), ())), preferred_element_type=jnp.float32)
    a = a_ref[0]
    tmp = jnp.dot(a, s, preferred_element_type=jnp.float32)
    diag_ref[0] = jnp.sum(s * tmp, axis=0, keepdims=True)


def _tc_pool(x16, a16, w, b2):
    out, diag = pl.pallas_call(
        _tc_pool_body,
        grid=(N_SUB,),
        in_specs=[
            pl.BlockSpec((1, GROUP, FDIM), lambda i: (i, 0, 0)),
            pl.BlockSpec((1, GROUP, GROUP), lambda i: (i, 0, 0)),
            pl.BlockSpec((FDIM, GROUP), lambda i: (0, 0)),
            pl.BlockSpec((1, GROUP), lambda i: (0, 0)),
        ],
        out_specs=[
            pl.BlockSpec((1, GROUP, FDIM), lambda i: (i, 0, 0)),
            pl.BlockSpec((1, 1, GROUP), lambda i: (i, 0, 0)),
        ],
        out_shape=[
            jax.ShapeDtypeStruct((N_SUB, GROUP, FDIM), jnp.float32),
            jax.ShapeDtypeStruct((N_SUB, 1, GROUP), jnp.float32),
        ],
    )(x16, a16, w, b2)
    return out, diag


def kernel(temporal_graph, temporal_adj, W_pool, b_pool):
    x16 = temporal_graph.reshape(N_SUB, GROUP, FDIM)
    src = temporal_adj[0].reshape(1, NEDGE).astype(jnp.int32)
    dst = temporal_adj[1].reshape(1, NEDGE).astype(jnp.int32)

    a_flat = _sc_build_adj(src, dst)
    a16 = a_flat[:A_SIZE].reshape(N_SUB, GROUP, GROUP)

    out, diag = _tc_pool(x16, a16, W_pool, b_pool.reshape(1, GROUP))

    temporal_pooled = out.reshape(1, NNODES, FDIM)
    new_weights = diag.reshape(NNODES)
    ar = jnp.arange(NNODES, dtype=temporal_adj.dtype)
    new_adj = jnp.stack([ar, ar])
    return (temporal_pooled, new_adj, new_weights)
